# speedup vs baseline: 19.2086x; 19.2086x over previous
import functools

import jax
import jax.numpy as jnp
from jax.experimental import pallas as pl
from jax.experimental.pallas import tpu as pltpu

_GAMMA = 0.01
_ALPHA = 0.5
_INF = 1e8


def _rollr1(a):
    return jnp.concatenate([a[:, -1:], a[:, :-1]], axis=1)


def _rolll1(a):
    return jnp.concatenate([a[:, 1:], a[:, :1]], axis=1)


def _dilate_kernel(t_ref, xw_ref, o1_ref, o2_ref, rs_ref, *, L, Bc):
    nd = 2 * L - 1
    ig = 1.0 / _GAMMA
    invl2 = 1.0 / float(L * L)

    t = t_ref[...]
    ii = jax.lax.broadcasted_iota(jnp.int32, (Bc, L), 1)
    tsh = _rolll1(t)

    wbuf0 = xw_ref[...]
    dd0 = (t - wbuf0[:, :L]) ** 2
    r0 = jnp.where(ii == 0, dd0, _INF)
    rs_ref[0] = r0

    def fwd(d, carry):
        wbuf, rp, r2u = carry
        wbuf = _rollr1(wbuf)
        dd = (t - wbuf[:, :L]) ** 2
        r1u = jnp.where(ii == 0, _INF, _rollr1(rp))
        m = jnp.minimum(jnp.minimum(r2u, r1u), rp)
        s = (jnp.exp((m - r2u) * ig) + jnp.exp((m - r1u) * ig)
             + jnp.exp((m - rp) * ig))
        r = m - _GAMMA * jnp.log(s)
        valid = (ii <= d) & (ii >= d - (L - 1))
        rnew = jnp.where(valid, dd + r, _INF)
        rs_ref[d] = rnew
        return wbuf, rnew, r1u

    inf_row = jnp.full((Bc, L), _INF, jnp.float32)
    wbuf_f, r_last, _ = jax.lax.fori_loop(1, nd, fwd, (wbuf0, r0, inf_row))

    s1 = jnp.sum(r_last[:, L - 1:L])

    e1_0 = jnp.where(ii == L - 1, 1.0, 0.0).astype(jnp.float32)
    zero_row = jnp.zeros((Bc, L), jnp.float32)

    def bwd(k, carry):
        vbuf, w1, e1, e2s, rn1, rn2s, acc = carry
        d = nd - 2 - k
        vbuf = _rolll1(vbuf)
        w0 = vbuf[:, :L]
        rc = rs_ref[d]
        da = (tsh - w0) ** 2
        db = (t - w1) ** 2
        dc = (tsh - w1) ** 2
        e1s = jnp.where(ii == L - 1, 0.0, _rolll1(e1))
        rn1s = _rolll1(rn1)
        wa = jnp.exp(jnp.minimum(rn1s - rc - da, 0.0) * ig)
        wb = jnp.exp(jnp.minimum(rn1 - rc - db, 0.0) * ig)
        wc = jnp.exp(jnp.minimum(rn2s - rc - dc, 0.0) * ig)
        valid = (ii <= d) & (ii >= d - (L - 1))
        ma = valid & (ii < L - 1)
        mb = valid & (ii >= d - (L - 2))
        mc = ma & (ii >= d - (L - 2))
        enew = (jnp.where(ma, wa * e1s, 0.0)
                + jnp.where(mb, wb * e1, 0.0)
                + jnp.where(mc, wc * e2s, 0.0))
        u = (2 * ii - d).astype(jnp.float32)
        acc = acc + enew * (u * u * invl2)
        return vbuf, w0, enew, e1s, rc, rn1s, acc

    carry0 = (wbuf_f, wbuf_f[:, :L], e1_0, zero_row, r_last, inf_row,
              zero_row)
    out = jax.lax.fori_loop(0, nd - 1, bwd, carry0)
    acc = out[-1]
    s2 = jnp.sum(acc)

    o1_ref[...] = jnp.full((1, 8, 128), s1, jnp.float32)
    o2_ref[...] = jnp.full((1, 8, 128), s2, jnp.float32)


def _build(L, Bc, nc, interpret=False):
    kern = functools.partial(_dilate_kernel, L=L, Bc=Bc)
    return pl.pallas_call(
        kern,
        grid=(nc,),
        in_specs=[pl.BlockSpec((Bc, L), lambda c: (c, 0)),
                  pl.BlockSpec((Bc, 3 * L), lambda c: (c, 0))],
        out_specs=[pl.BlockSpec((1, 8, 128), lambda c: (c, 0, 0)),
                   pl.BlockSpec((1, 8, 128), lambda c: (c, 0, 0))],
        out_shape=[jax.ShapeDtypeStruct((nc, 8, 128), jnp.float32),
                   jax.ShapeDtypeStruct((nc, 8, 128), jnp.float32)],
        scratch_shapes=[pltpu.VMEM((2 * L - 1, Bc, L), jnp.float32)],
        compiler_params=pltpu.CompilerParams(
            dimension_semantics=("parallel",)),
        interpret=interpret,
    )


@jax.jit
def kernel(input, target):
    B, L, _ = input.shape
    x = input[:, :, 0].astype(jnp.float32)
    t = target[:, :, 0].astype(jnp.float32)
    Bc = 32 if B % 32 == 0 else B
    nc = B // Bc
    xpad = jnp.zeros((B, 3 * L), jnp.float32)
    xpad = xpad.at[:, L - 1:2 * L - 1].set(x[:, ::-1])
    xw = jnp.roll(xpad, -(2 * L - 2), axis=1)
    o1, o2 = _build(L, Bc, nc)(t, xw)
    s1 = jnp.sum(o1[:, 0, 0])
    s2 = jnp.sum(o2[:, 0, 0])
    return _ALPHA * (s1 / B) + (1.0 - _ALPHA) * (s2 / B)


# speedup vs baseline: 21.6269x; 1.1259x over previous
import functools

import jax
import jax.numpy as jnp
from jax.experimental import pallas as pl
from jax.experimental.pallas import tpu as pltpu

_GAMMA = 0.01
_ALPHA = 0.5
_INF = 1e8


def _rollr1(a):
    return jnp.concatenate([a[:, -1:], a[:, :-1]], axis=1)


def _rolll1(a):
    return jnp.concatenate([a[:, 1:], a[:, :1]], axis=1)


def _dilate_kernel(t_ref, xw_ref, o1_ref, o2_ref, rs_ref, ds_ref, dss_ref,
                   *, L, Bc):
    nd = 2 * L - 1
    ig = 1.0 / _GAMMA
    invl2 = 1.0 / float(L * L)

    t = t_ref[...]
    ii = jax.lax.broadcasted_iota(jnp.int32, (Bc, L), 1)
    tsh = _rolll1(t)

    xw0 = xw_ref[...]
    w0 = xw0[:, :L]
    dd0 = (t - w0) ** 2
    ds_ref[0] = dd0
    dss_ref[1] = (tsh - w0) ** 2
    r0 = jnp.where(ii == 0, dd0, _INF)
    rs_ref[0] = r0

    def fwd(d, carry):
        rp, r2u = carry
        w = pltpu.roll(xw0, d, axis=1)[:, :L]
        dd = (t - w) ** 2
        ds_ref[d] = dd
        dss_ref[d + 1] = (tsh - w) ** 2
        r1u = jnp.where(ii == 0, _INF, _rollr1(rp))
        m = jnp.minimum(jnp.minimum(r2u, r1u), rp)
        s = (jnp.exp((m - r2u) * ig) + jnp.exp((m - r1u) * ig)
             + jnp.exp((m - rp) * ig))
        r = m - _GAMMA * jnp.log(s)
        valid = (ii <= d) & (ii >= d - (L - 1))
        rnew = jnp.where(valid, dd + r, _INF)
        rs_ref[d] = rnew
        return rnew, r1u

    inf_row = jnp.full((Bc, L), _INF, jnp.float32)
    r_last, _ = jax.lax.fori_loop(1, nd, fwd, (r0, inf_row))

    s1 = jnp.sum(r_last[:, L - 1:L])

    e1_0 = jnp.where(ii == L - 1, 1.0, 0.0).astype(jnp.float32)
    zero_row = jnp.zeros((Bc, L), jnp.float32)

    def bwd(k, carry):
        e1, e2s, g1, g2, acc = carry
        d = nd - 2 - k
        rc = rs_ref[d]
        rn1 = rs_ref[d + 1]
        da = dss_ref[d + 1]
        db = ds_ref[d + 1]
        dc = dss_ref[d + 2]
        e1s = jnp.where(ii == L - 1, 0.0, _rolll1(e1))
        wa = jnp.exp(jnp.minimum(g1 - rc - da, 0.0) * ig)
        wb = jnp.exp(jnp.minimum(rn1 - rc - db, 0.0) * ig)
        wc = jnp.exp(jnp.minimum(g2 - rc - dc, 0.0) * ig)
        valid = (ii <= d) & (ii >= d - (L - 1))
        ma = valid & (ii < L - 1)
        mb = valid & (ii >= d - (L - 2))
        mc = ma & (ii >= d - (L - 2))
        enew = (jnp.where(ma, wa * e1s, 0.0)
                + jnp.where(mb, wb * e1, 0.0)
                + jnp.where(mc, wc * e2s, 0.0))
        u = (2 * ii - d).astype(jnp.float32)
        acc = acc + enew * (u * u * invl2)
        gnew = _rolll1(rc)
        return enew, e1s, gnew, g1, acc

    g1_0 = _rolll1(r_last)
    carry0 = (e1_0, zero_row, g1_0, inf_row, zero_row)
    out = jax.lax.fori_loop(0, nd - 1, bwd, carry0)
    acc = out[-1]
    s2 = jnp.sum(acc)

    o1_ref[...] = jnp.full((1, 8, 128), s1, jnp.float32)
    o2_ref[...] = jnp.full((1, 8, 128), s2, jnp.float32)


def _build(L, Bc, nc, interpret=False):
    kern = functools.partial(_dilate_kernel, L=L, Bc=Bc)
    return pl.pallas_call(
        kern,
        grid=(nc,),
        in_specs=[pl.BlockSpec((Bc, L), lambda c: (c, 0)),
                  pl.BlockSpec((Bc, 3 * L), lambda c: (c, 0))],
        out_specs=[pl.BlockSpec((1, 8, 128), lambda c: (c, 0, 0)),
                   pl.BlockSpec((1, 8, 128), lambda c: (c, 0, 0))],
        out_shape=[jax.ShapeDtypeStruct((nc, 8, 128), jnp.float32),
                   jax.ShapeDtypeStruct((nc, 8, 128), jnp.float32)],
        scratch_shapes=[pltpu.VMEM((2 * L - 1, Bc, L), jnp.float32),
                        pltpu.VMEM((2 * L - 1, Bc, L), jnp.float32),
                        pltpu.VMEM((2 * L + 1, Bc, L), jnp.float32)],
        compiler_params=pltpu.CompilerParams(
            dimension_semantics=("parallel",)),
        interpret=interpret,
    )


@jax.jit
def kernel(input, target):
    B, L, _ = input.shape
    x = input[:, :, 0].astype(jnp.float32)
    t = target[:, :, 0].astype(jnp.float32)
    Bc = 32 if B % 32 == 0 else B
    nc = B // Bc
    xpad = jnp.zeros((B, 3 * L), jnp.float32)
    xpad = xpad.at[:, L - 1:2 * L - 1].set(x[:, ::-1])
    xw = jnp.roll(xpad, -(2 * L - 2), axis=1)
    o1, o2 = _build(L, Bc, nc)(t, xw)
    s1 = jnp.sum(o1[:, 0, 0])
    s2 = jnp.sum(o2[:, 0, 0])
    return _ALPHA * (s1 / B) + (1.0 - _ALPHA) * (s2 / B)
